# Initial kernel scaffold; baseline (speedup 1.0000x reference)
import jax
import jax.numpy as jnp
from jax import lax
from jax.experimental import pallas as pl
from jax.experimental.pallas import tpu as pltpu
from jax.experimental.pallas import tpu_sc as plsc

F32 = jnp.float32
I32 = jnp.int32

NE = 50000
NU = 50000
NI = 30000
EDG = 800000
D = 64
H = 32
NREL = 32

NTILE = 16
KCH = 1024
SUB = 128
NSUB = KCH // SUB
NCHUNK = 50
E_PAD = NTILE * KCH * NCHUNK
IDXW = 128
IDXROWS = E_PAD // IDXW

RP_ENT = 50176
Q_ENT = 3136
RP_ITM = 30208
Q_ITM = 1888


def _pad_idx(a, pad_vals):
    return jnp.concatenate([a.astype(I32), pad_vals]).reshape(IDXROWS, IDXW)


def _make_sc_kernel(r_pad, q, wch, use_w, mean):
    mesh = plsc.VectorSubcoreMesh(core_axis_name="c", subcore_axis_name="s")
    out_type = (jax.ShapeDtypeStruct((r_pad, H), F32),
                jax.ShapeDtypeStruct((r_pad, H), F32))
    scratch = [
        pltpu.VMEM((NSUB, IDXW), I32),
        pltpu.VMEM((NSUB, IDXW), I32),
        pltpu.VMEM((KCH, H), F32),
        pltpu.VMEM_SHARED((r_pad, H), F32),
        pltpu.SemaphoreType.DMA,
    ]
    if use_w:
        scratch += [pltpu.VMEM((NSUB, IDXW), I32),
                    pltpu.VMEM((KCH, H), F32),
                    pltpu.VMEM_SHARED((NTILE * NREL, H), F32)]
    if mean:
        scratch += [pltpu.VMEM_SHARED((r_pad,), F32),
                    pltpu.VMEM((SUB,), F32),
                    pltpu.VMEM((wch, H), F32),
                    pltpu.VMEM((wch,), F32)]

    def body(*refs):
        src_lo, src_hi, gidx, sidx = refs[0], refs[1], refs[2], refs[3]
        i = 4
        if use_w:
            tyidx, w_lo, w_hi = refs[i], refs[i + 1], refs[i + 2]
            i += 3
        zrows = refs[i]
        i += 1
        if mean:
            zcnt, ones_h = refs[i], refs[i + 1]
            i += 2
        out_lo, out_hi = refs[i], refs[i + 1]
        i += 2
        g_v, s_v, rows_v, acc, sem = (refs[i], refs[i + 1], refs[i + 2],
                                      refs[i + 3], refs[i + 4])
        i += 5
        if use_w:
            ty_v, wrows_v, w_sp = refs[i], refs[i + 1], refs[i + 2]
            i += 3
        if mean:
            cnt, ones_v, buf_v, cbuf_v = (refs[i], refs[i + 1], refs[i + 2],
                                          refs[i + 3])

        c = lax.axis_index("c")

        def run(src, wtab, out):
            t = lax.axis_index("s")
            pltpu.sync_copy(zrows, acc.at[pl.ds(t * q, q)])
            if mean:
                pltpu.sync_copy(zcnt, cnt.at[pl.ds(t * q, q)])
                pltpu.sync_copy(ones_h, ones_v)
            if use_w:
                pltpu.sync_copy(wtab, w_sp.at[pl.ds(t * NREL, NREL)])
            plsc.subcore_barrier()

            def chunk(ci, carry):
                rb = t * (NCHUNK * NSUB) + ci * NSUB
                pltpu.sync_copy(gidx.at[pl.ds(rb, NSUB)], g_v)
                pltpu.sync_copy(sidx.at[pl.ds(rb, NSUB)], s_v)
                if use_w:
                    pltpu.sync_copy(tyidx.at[pl.ds(rb, NSUB)], ty_v)
                    toff = t * NREL
                    for r in range(NSUB):
                        for g8 in range(IDXW // 16):
                            sl = (r, pl.ds(g8 * 16, 16))
                            ty_v[sl] = ty_v[sl] + toff
                descs = [pltpu.async_copy(src.at[g_v.at[j]],
                                          rows_v.at[pl.ds(j * SUB, SUB)], sem)
                         for j in range(NSUB)]
                if use_w:
                    descs += [pltpu.async_copy(w_sp.at[ty_v.at[j]],
                                               wrows_v.at[pl.ds(j * SUB, SUB)],
                                               sem)
                              for j in range(NSUB)]
                for dsc in descs:
                    dsc.wait()
                if use_w:
                    def mul8(m, cr):
                        for r in range(8):
                            rw = m * 8 + r
                            for hh in range(2):
                                sl = (rw, pl.ds(hh * 16, 16))
                                rows_v[sl] = rows_v[sl] * wrows_v[sl]
                        return cr
                    lax.fori_loop(0, KCH // 8, mul8, 0)
                descs = [pltpu.async_copy(rows_v.at[pl.ds(j * SUB, SUB)],
                                          acc.at[s_v.at[j]], sem, add=True)
                         for j in range(NSUB)]
                if mean:
                    descs += [pltpu.async_copy(ones_v, cnt.at[s_v.at[j]], sem,
                                               add=True)
                              for j in range(NSUB)]
                for dsc in descs:
                    dsc.wait()
                return carry

            lax.fori_loop(0, NCHUNK, chunk, 0)
            plsc.subcore_barrier()

            if mean:
                iota16 = lax.iota(I32, 16)
                for wc in range(q // wch):
                    rbase = t * q + wc * wch
                    pltpu.sync_copy(acc.at[pl.ds(rbase, wch)], buf_v)
                    pltpu.sync_copy(cnt.at[pl.ds(rbase, wch)], cbuf_v)

                    def div16(g, cr):
                        g0 = g * 16
                        c16 = cbuf_v[pl.ds(g0, 16)]
                        cmax = jnp.maximum(c16, 1.0)
                        rows16 = g0 + iota16
                        for ccol in range(H):
                            colv = jnp.full((16,), ccol, I32)
                            v = plsc.load_gather(buf_v, [rows16, colv])
                            plsc.store_scatter(buf_v, [rows16, colv], v / cmax)
                        return cr

                    lax.fori_loop(0, wch // 16, div16, 0)
                    pltpu.sync_copy(buf_v, out.at[pl.ds(rbase, wch)])
            else:
                pltpu.sync_copy(acc.at[pl.ds(t * q, q)], out.at[pl.ds(t * q, q)])

        @pl.when(c == 0)
        def _():
            run(src_lo, w_lo if use_w else None, out_lo)

        @pl.when(c == 1)
        def _():
            run(src_hi, w_hi if use_w else None, out_hi)

    return pl.kernel(body, out_type=out_type, mesh=mesh, scratch_types=scratch)


def _fusion_call(kg_lo, kg_hi, in_lo, in_hi, w1, w2):
    B = 500
    grid = (NI // B,)

    def fbody(kl, kh, il, ih, w1r, w2r, fus, fl, fh, kc, ic):
        kg = jnp.concatenate([kl[...], kh[...]], axis=1)
        it = jnp.concatenate([il[...], ih[...]], axis=1)
        z = lax.dot_general(kg, w1r[...], (((1,), (1,)), ((), ())),
                            preferred_element_type=F32)
        z = z + lax.dot_general(it, w2r[...], (((1,), (1,)), ((), ())),
                                preferred_element_type=F32)
        g = jax.nn.sigmoid(z)
        f = g * kg + (1.0 - g) * it
        fus[...] = f
        fl[...] = f[:, :H]
        fh[...] = f[:, H:]
        kc[...] = kg
        ic[...] = it

    bs_h = pl.BlockSpec((B, H), lambda i: (i, 0))
    bs_d = pl.BlockSpec((B, D), lambda i: (i, 0))
    bs_w = pl.BlockSpec((D, D), lambda i: (0, 0))
    return pl.pallas_call(
        fbody, grid=grid,
        in_specs=[bs_h, bs_h, bs_h, bs_h, bs_w, bs_w],
        out_specs=[bs_d, bs_h, bs_h, bs_d, bs_d],
        out_shape=[jax.ShapeDtypeStruct((NI, D), F32),
                   jax.ShapeDtypeStruct((NI, H), F32),
                   jax.ShapeDtypeStruct((NI, H), F32),
                   jax.ShapeDtypeStruct((NI, D), F32),
                   jax.ShapeDtypeStruct((NI, D), F32)],
    )(kg_lo, kg_hi, in_lo, in_hi, w1, w2)


def kernel(entity_emb, user_emb, edge_index, edge_type, interact_mat, weight,
           W1, W2):
    head = edge_index[0]
    tail = edge_index[1]
    row = interact_mat[0]
    col = interact_mat[1]
    ent_lo = entity_emb[:, :H]
    ent_hi = entity_emb[:, H:]
    usr_lo = user_emb[:, :H]
    usr_hi = user_emb[:, H:]
    w_lo = weight[:, :H]
    w_hi = weight[:, H:]

    npad = E_PAD - EDG
    zpad = jnp.zeros((npad,), I32)
    trash_ent = (jnp.arange(npad, dtype=I32) % (RP_ENT - NE)) + NE
    trash_itm = (jnp.arange(npad, dtype=I32) % (RP_ITM - NI)) + NI

    head_p = _pad_idx(head, trash_ent)
    tail_p = _pad_idx(tail, zpad)
    type_p = _pad_idx(edge_type, zpad)
    rowg_p = _pad_idx(row, zpad)
    row_p = _pad_idx(row, trash_ent)
    colg_p = _pad_idx(col, zpad)
    col_p = _pad_idx(col, trash_itm)

    ones128 = jnp.ones((SUB,), F32)
    z_ent_rows = jnp.zeros((Q_ENT, H), F32)
    z_ent_cnt = jnp.zeros((Q_ENT,), F32)
    z_itm_rows = jnp.zeros((Q_ITM, H), F32)
    z_itm_cnt = jnp.zeros((Q_ITM,), F32)

    kg_k = _make_sc_kernel(RP_ENT, Q_ENT, 784, True, True)
    kg_lo, kg_hi = kg_k(ent_lo, ent_hi, tail_p, head_p, type_p, w_lo, w_hi,
                        z_ent_rows, z_ent_cnt, ones128)

    int_k = _make_sc_kernel(RP_ITM, Q_ITM, 944, False, True)
    int_lo, int_hi = int_k(usr_lo, usr_hi, rowg_p, col_p,
                           z_itm_rows, z_itm_cnt, ones128)

    fus, fus_lo, fus_hi, kg_cat, int_cat = _fusion_call(
        kg_lo[:NI], kg_hi[:NI], int_lo[:NI], int_hi[:NI], W1, W2)

    usr_k = _make_sc_kernel(RP_ENT, Q_ENT, 784, False, False)
    ua_lo, ua_hi = usr_k(fus_lo, fus_hi, colg_p, row_p, z_ent_rows)

    att = jnp.concatenate([kg_lo[NI:NE], kg_hi[NI:NE]], axis=1)
    final_entity = jnp.concatenate([fus, att], axis=0)
    user_agg = jnp.concatenate([ua_lo[:NU], ua_hi[:NU]], axis=1)
    return final_entity, user_agg, kg_cat, int_cat


# speedup vs baseline: 5.1533x; 5.1533x over previous
import jax
import jax.numpy as jnp
from jax import lax
from jax.experimental import pallas as pl
from jax.experimental.pallas import tpu as pltpu
from jax.experimental.pallas import tpu_sc as plsc

F32 = jnp.float32
I32 = jnp.int32

NE = 50000
NU = 50000
NI = 30000
EDG = 800000
D = 64
H = 32
NREL = 32

NTILE = 16
SUB = 128
E_PAD = 819200
EPT = E_PAD // NTILE
IDXW = 128
IDXROWS = E_PAD // IDXW

RP_ENT = 50176
Q_ENT = 3136
RP_ITM = 30208
Q_ITM = 1888


def _pad_idx(a, pad_vals):
    return jnp.concatenate([a.astype(I32), pad_vals]).reshape(IDXROWS, IDXW)


def _make_sc_kernel(r_pad, q, kch, use_w, mean):
    mesh = plsc.VectorSubcoreMesh(core_axis_name="c", subcore_axis_name="s")
    out_type = [jax.ShapeDtypeStruct((r_pad, H), F32),
                jax.ShapeDtypeStruct((r_pad, H), F32)]
    if mean:
        out_type.append(jax.ShapeDtypeStruct((r_pad,), F32))
    nsub = kch // SUB
    nchunk = EPT // kch
    scratch = [
        pltpu.VMEM((nsub, IDXW), I32),
        pltpu.VMEM((nsub, IDXW), I32),
        pltpu.VMEM((kch, H), F32),
        pltpu.VMEM_SHARED((r_pad, H), F32),
        pltpu.SemaphoreType.DMA,
    ]
    if use_w:
        scratch += [pltpu.VMEM((nsub, IDXW), I32),
                    pltpu.VMEM((kch, H), F32)]
    if mean:
        scratch += [pltpu.VMEM_SHARED((r_pad,), F32),
                    pltpu.VMEM((SUB,), F32)]

    def body(*refs):
        src_lo, src_hi, gidx, sidx = refs[0], refs[1], refs[2], refs[3]
        i = 4
        if use_w:
            tyidx, w_lo, w_hi = refs[i], refs[i + 1], refs[i + 2]
            i += 3
        zrows = refs[i]
        i += 1
        if mean:
            zcnt, ones_h = refs[i], refs[i + 1]
            i += 2
        out_lo, out_hi = refs[i], refs[i + 1]
        i += 2
        if mean:
            out_cnt = refs[i]
            i += 1
        g_v, s_v, rows_v, acc, sem = (refs[i], refs[i + 1], refs[i + 2],
                                      refs[i + 3], refs[i + 4])
        i += 5
        if use_w:
            ty_v, wrows_v = refs[i], refs[i + 1]
            i += 2
        if mean:
            cnt, ones_v = refs[i], refs[i + 1]

        c = lax.axis_index("c")

        def run(src, wtab, out, write_cnt):
            t = lax.axis_index("s")
            pltpu.sync_copy(zrows, acc.at[pl.ds(t * q, q)])
            if mean:
                @pl.when(t == 0)
                def _():
                    pltpu.sync_copy(zcnt, cnt)
                pltpu.sync_copy(ones_h, ones_v)
            plsc.subcore_barrier()

            def chunk(ci, carry):
                rb = t * (nchunk * nsub) + ci * nsub
                pltpu.sync_copy(gidx.at[pl.ds(rb, nsub)], g_v)
                pltpu.sync_copy(sidx.at[pl.ds(rb, nsub)], s_v)
                if use_w:
                    pltpu.sync_copy(tyidx.at[pl.ds(rb, nsub)], ty_v)
                    toff = t * NREL
                    for r in range(nsub):
                        for g8 in range(IDXW // 16):
                            sl = (r, pl.ds(g8 * 16, 16))
                            ty_v[sl] = ty_v[sl] + toff
                descs = [pltpu.async_copy(src.at[g_v.at[j]],
                                          rows_v.at[pl.ds(j * SUB, SUB)], sem)
                         for j in range(nsub)]
                if use_w:
                    descs += [pltpu.async_copy(wtab.at[ty_v.at[j]],
                                               wrows_v.at[pl.ds(j * SUB, SUB)],
                                               sem)
                              for j in range(nsub)]
                for dsc in descs:
                    dsc.wait()
                if use_w:
                    def mul8(m, cr):
                        for r in range(8):
                            rw = m * 8 + r
                            for hh in range(2):
                                sl = (rw, pl.ds(hh * 16, 16))
                                rows_v[sl] = rows_v[sl] * wrows_v[sl]
                        return cr
                    lax.fori_loop(0, kch // 8, mul8, 0)
                descs = [pltpu.async_copy(rows_v.at[pl.ds(j * SUB, SUB)],
                                          acc.at[s_v.at[j]], sem, add=True)
                         for j in range(nsub)]
                if mean:
                    descs += [pltpu.async_copy(ones_v, cnt.at[s_v.at[j]], sem,
                                               add=True)
                              for j in range(nsub)]
                for dsc in descs:
                    dsc.wait()
                return carry

            lax.fori_loop(0, nchunk, chunk, 0)
            plsc.subcore_barrier()
            pltpu.sync_copy(acc.at[pl.ds(t * q, q)], out.at[pl.ds(t * q, q)])
            if mean and write_cnt:
                @pl.when(t == 0)
                def _():
                    pltpu.sync_copy(cnt, out_cnt)

        @pl.when(c == 0)
        def _():
            run(src_lo, w_lo if use_w else None, out_lo, True)

        @pl.when(c == 1)
        def _():
            run(src_hi, w_hi if use_w else None, out_hi, False)

    return pl.kernel(body, out_type=tuple(out_type), mesh=mesh,
                     scratch_types=scratch,
                     compiler_params=pltpu.CompilerParams(
                         use_tc_tiling_on_sc=False))


def _fusion_call(kg_lo, kg_hi, kg_cnt, in_lo, in_hi, in_cnt, w1, w2):
    B = 600
    grid = (NI // B,)

    def fbody(kl, kh, kcn, il, ih, icn, w1r, w2r, fus, fl, fh, kc, ic):
        kg = jnp.concatenate([kl[...], kh[...]], axis=1)
        kg = kg / jnp.maximum(kcn[...], 1.0)
        it = jnp.concatenate([il[...], ih[...]], axis=1)
        it = it / jnp.maximum(icn[...], 1.0)
        z = lax.dot_general(kg, w1r[...], (((1,), (1,)), ((), ())),
                            preferred_element_type=F32)
        z = z + lax.dot_general(it, w2r[...], (((1,), (1,)), ((), ())),
                                preferred_element_type=F32)
        g = jax.nn.sigmoid(z)
        f = g * kg + (1.0 - g) * it
        fus[...] = f
        fl[...] = f[:, :H]
        fh[...] = f[:, H:]
        kc[...] = kg
        ic[...] = it

    bs_h = pl.BlockSpec((B, H), lambda i: (i, 0))
    bs_d = pl.BlockSpec((B, D), lambda i: (i, 0))
    bs_c = pl.BlockSpec((B, 1), lambda i: (i, 0))
    bs_w = pl.BlockSpec((D, D), lambda i: (0, 0))
    return pl.pallas_call(
        fbody, grid=grid,
        in_specs=[bs_h, bs_h, bs_c, bs_h, bs_h, bs_c, bs_w, bs_w],
        out_specs=[bs_d, bs_h, bs_h, bs_d, bs_d],
        out_shape=[jax.ShapeDtypeStruct((NI, D), F32),
                   jax.ShapeDtypeStruct((NI, H), F32),
                   jax.ShapeDtypeStruct((NI, H), F32),
                   jax.ShapeDtypeStruct((NI, D), F32),
                   jax.ShapeDtypeStruct((NI, D), F32)],
    )(kg_lo, kg_hi, kg_cnt, in_lo, in_hi, in_cnt, w1, w2)


def _att_div_call(kg_lo, kg_hi, kg_cnt):
    NA = NE - NI
    B = 400
    grid = (NA // B,)

    def abody(kl, kh, kcn, out):
        kg = jnp.concatenate([kl[...], kh[...]], axis=1)
        out[...] = kg / jnp.maximum(kcn[...], 1.0)

    return pl.pallas_call(
        abody, grid=grid,
        in_specs=[pl.BlockSpec((B, H), lambda i: (i, 0)),
                  pl.BlockSpec((B, H), lambda i: (i, 0)),
                  pl.BlockSpec((B, 1), lambda i: (i, 0))],
        out_specs=pl.BlockSpec((B, D), lambda i: (i, 0)),
        out_shape=jax.ShapeDtypeStruct((NA, D), F32),
    )(kg_lo, kg_hi, kg_cnt)


def kernel(entity_emb, user_emb, edge_index, edge_type, interact_mat, weight,
           W1, W2):
    head = edge_index[0]
    tail = edge_index[1]
    row = interact_mat[0]
    col = interact_mat[1]
    ent_lo = entity_emb[:, :H]
    ent_hi = entity_emb[:, H:]
    usr_lo = user_emb[:, :H]
    usr_hi = user_emb[:, H:]
    w_lo = jnp.tile(weight[:, :H], (NTILE, 1))
    w_hi = jnp.tile(weight[:, H:], (NTILE, 1))

    npad = E_PAD - EDG
    zpad = jnp.zeros((npad,), I32)
    trash_ent = (jnp.arange(npad, dtype=I32) % (RP_ENT - NE)) + NE
    trash_itm = (jnp.arange(npad, dtype=I32) % (RP_ITM - NI)) + NI

    head_p = _pad_idx(head, trash_ent)
    tail_p = _pad_idx(tail, zpad)
    type_p = _pad_idx(edge_type, zpad)
    rowg_p = _pad_idx(row, zpad)
    row_p = _pad_idx(row, trash_ent)
    colg_p = _pad_idx(col, zpad)
    col_p = _pad_idx(col, trash_itm)

    ones128 = jnp.ones((SUB,), F32)
    z_ent_rows = jnp.zeros((Q_ENT, H), F32)
    z_ent_cnt = jnp.zeros((RP_ENT,), F32)
    z_itm_rows = jnp.zeros((Q_ITM, H), F32)
    z_itm_cnt = jnp.zeros((RP_ITM,), F32)

    kg_k = _make_sc_kernel(RP_ENT, Q_ENT, 256, True, True)
    kg_lo, kg_hi, kg_cnt = kg_k(ent_lo, ent_hi, tail_p, head_p, type_p,
                                w_lo, w_hi, z_ent_rows, z_ent_cnt, ones128)

    int_k = _make_sc_kernel(RP_ITM, Q_ITM, 1024, False, True)
    int_lo, int_hi, int_cnt = int_k(usr_lo, usr_hi, rowg_p, col_p,
                                    z_itm_rows, z_itm_cnt, ones128)

    fus, fus_lo, fus_hi, kg_cat, int_cat = _fusion_call(
        kg_lo[:NI], kg_hi[:NI], kg_cnt[:NI, None],
        int_lo[:NI], int_hi[:NI], int_cnt[:NI, None], W1, W2)

    usr_k = _make_sc_kernel(RP_ENT, Q_ENT, 512, False, False)
    ua_lo, ua_hi = usr_k(fus_lo, fus_hi, colg_p, row_p, z_ent_rows)

    att = _att_div_call(kg_lo[NI:NE], kg_hi[NI:NE], kg_cnt[NI:NE, None])
    final_entity = jnp.concatenate([fus, att], axis=0)
    user_agg = jnp.concatenate([ua_lo[:NU], ua_hi[:NU]], axis=1)
    return final_entity, user_agg, kg_cat, int_cat


# speedup vs baseline: 6.6412x; 1.2887x over previous
import jax
import jax.numpy as jnp
from jax import lax
from jax.experimental import pallas as pl
from jax.experimental.pallas import tpu as pltpu
from jax.experimental.pallas import tpu_sc as plsc

F32 = jnp.float32
I32 = jnp.int32

NE = 50000
NU = 50000
NI = 30000
EDG = 800000
D = 64
H = 32
NREL = 32

NTILE = 16
SUB = 128
E_PAD = 819200
EPT = E_PAD // NTILE
IDXW = 128
IDXROWS = E_PAD // IDXW

RP_ENT = 50176
Q_ENT = 3136
RP_ITM = 30208
Q_ITM = 1888


def _pad_idx(a, pad_vals):
    return jnp.concatenate([a.astype(I32), pad_vals]).reshape(IDXROWS, IDXW)


def _make_sc_kernel(r_pad, q, kch, grp, use_w, mean):
    mesh = plsc.VectorSubcoreMesh(core_axis_name="c", subcore_axis_name="s")
    out_type = [jax.ShapeDtypeStruct((r_pad, H), F32),
                jax.ShapeDtypeStruct((r_pad, H), F32)]
    if mean:
        out_type.append(jax.ShapeDtypeStruct((r_pad,), F32))
    nsub = kch // SUB
    nchunk = EPT // kch
    gn = grp * nsub
    assert grp % 2 == 0 and nchunk % grp == 0
    ngroups = nchunk // grp
    scratch = [
        pltpu.VMEM((gn, IDXW), I32),
        pltpu.VMEM((gn, IDXW), I32),
        pltpu.VMEM((kch, H), F32),
        pltpu.VMEM((kch, H), F32),
        pltpu.VMEM_SHARED((r_pad, H), F32),
        pltpu.SemaphoreType.DMA,
        pltpu.SemaphoreType.DMA,
        pltpu.SemaphoreType.DMA,
    ]
    if use_w:
        scratch += [pltpu.VMEM((gn, IDXW), I32),
                    pltpu.VMEM((kch, H), F32),
                    pltpu.VMEM((kch, H), F32)]
    if mean:
        scratch += [pltpu.VMEM_SHARED((r_pad,), F32),
                    pltpu.VMEM((SUB,), F32)]

    def body(*refs):
        src_lo, src_hi, gidx, sidx = refs[0], refs[1], refs[2], refs[3]
        i = 4
        if use_w:
            tyidx, w_lo, w_hi = refs[i], refs[i + 1], refs[i + 2]
            i += 3
        zrows = refs[i]
        i += 1
        if mean:
            zcnt, ones_h = refs[i], refs[i + 1]
            i += 2
        out_lo, out_hi = refs[i], refs[i + 1]
        i += 2
        if mean:
            out_cnt = refs[i]
            i += 1
        gi_v, si_v, rows0, rows1, acc, sem0, sem1, ssem = refs[i:i + 8]
        i += 8
        if use_w:
            ti_v, wrows0, wrows1 = refs[i], refs[i + 1], refs[i + 2]
            i += 3
        if mean:
            cnt, ones_v = refs[i], refs[i + 1]

        c = lax.axis_index("c")
        rows_b = (rows0, rows1)
        wrows_b = (wrows0, wrows1) if use_w else None
        sems = (sem0, sem1)

        def run(src, wtab, out, write_cnt):
            t = lax.axis_index("s")
            pltpu.sync_copy(zrows, acc.at[pl.ds(t * q, q)])
            if mean:
                @pl.when(t == 0)
                def _():
                    pltpu.sync_copy(zcnt, cnt)
                pltpu.sync_copy(ones_h, ones_v)
            plsc.subcore_barrier()

            def issue_g(ck, pb):
                for j in range(nsub):
                    r = ck * nsub + j
                    pltpu.async_copy(src.at[gi_v.at[r]],
                                     rows_b[pb].at[pl.ds(j * SUB, SUB)],
                                     sems[pb])
                    if use_w:
                        pltpu.async_copy(wtab.at[ti_v.at[r]],
                                         wrows_b[pb].at[pl.ds(j * SUB, SUB)],
                                         sems[pb])

            def wait_g(ck, pb):
                for j in range(nsub):
                    r = ck * nsub + j
                    pltpu.make_async_copy(
                        src.at[gi_v.at[r]],
                        rows_b[pb].at[pl.ds(j * SUB, SUB)], sems[pb]).wait()
                    if use_w:
                        pltpu.make_async_copy(
                            wtab.at[ti_v.at[r]],
                            wrows_b[pb].at[pl.ds(j * SUB, SUB)],
                            sems[pb]).wait()

            def do_chunk(ck, pb):
                wait_g(ck, pb)
                if use_w:
                    def mul8(m, cr):
                        for r8 in range(8):
                            rw = m * 8 + r8
                            for hh in range(2):
                                sl = (rw, pl.ds(hh * 16, 16))
                                rows_b[pb][sl] = rows_b[pb][sl] * wrows_b[pb][sl]
                        return cr
                    lax.fori_loop(0, kch // 8, mul8, 0)
                sds = []
                for j in range(nsub):
                    r = ck * nsub + j
                    sds.append(pltpu.async_copy(
                        rows_b[pb].at[pl.ds(j * SUB, SUB)],
                        acc.at[si_v.at[r]], ssem, add=True))
                    if mean:
                        sds.append(pltpu.async_copy(ones_v, cnt.at[si_v.at[r]],
                                                    ssem, add=True))
                for dd in sds:
                    dd.wait()

            def group(g, carry):
                base = t * (nchunk * nsub) + g * gn
                pltpu.sync_copy(gidx.at[pl.ds(base, gn)], gi_v)
                pltpu.sync_copy(sidx.at[pl.ds(base, gn)], si_v)
                if use_w:
                    pltpu.sync_copy(tyidx.at[pl.ds(base, gn)], ti_v)
                    toff = t * NREL
                    for r in range(gn):
                        for g8 in range(IDXW // 16):
                            sl = (r, pl.ds(g8 * 16, 16))
                            ti_v[sl] = ti_v[sl] + toff
                issue_g(0, 0)

                def pair(p, cr):
                    a = 2 * p
                    issue_g(a + 1, 1)
                    do_chunk(a, 0)

                    @pl.when(p < grp // 2 - 1)
                    def _():
                        issue_g(a + 2, 0)
                    do_chunk(a + 1, 1)
                    return cr

                lax.fori_loop(0, grp // 2, pair, 0)
                return carry

            lax.fori_loop(0, ngroups, group, 0)
            plsc.subcore_barrier()
            pltpu.sync_copy(acc.at[pl.ds(t * q, q)], out.at[pl.ds(t * q, q)])
            if mean and write_cnt:
                @pl.when(t == 0)
                def _():
                    pltpu.sync_copy(cnt, out_cnt)

        @pl.when(c == 0)
        def _():
            run(src_lo, w_lo if use_w else None, out_lo, True)

        @pl.when(c == 1)
        def _():
            run(src_hi, w_hi if use_w else None, out_hi, False)

    return pl.kernel(body, out_type=tuple(out_type), mesh=mesh,
                     scratch_types=scratch,
                     compiler_params=pltpu.CompilerParams(
                         use_tc_tiling_on_sc=False))


def _fusion_call(kg_lo, kg_hi, kg_cnt, in_lo, in_hi, in_cnt, w1, w2):
    B = 600
    grid = (NI // B,)

    def fbody(kl, kh, kcn, il, ih, icn, w1r, w2r, fus, fl, fh, kc, ic):
        kg = jnp.concatenate([kl[...], kh[...]], axis=1)
        kg = kg / jnp.maximum(kcn[...], 1.0)
        it = jnp.concatenate([il[...], ih[...]], axis=1)
        it = it / jnp.maximum(icn[...], 1.0)
        z = lax.dot_general(kg, w1r[...], (((1,), (1,)), ((), ())),
                            preferred_element_type=F32)
        z = z + lax.dot_general(it, w2r[...], (((1,), (1,)), ((), ())),
                                preferred_element_type=F32)
        g = jax.nn.sigmoid(z)
        f = g * kg + (1.0 - g) * it
        fus[...] = f
        fl[...] = f[:, :H]
        fh[...] = f[:, H:]
        kc[...] = kg
        ic[...] = it

    bs_h = pl.BlockSpec((B, H), lambda i: (i, 0))
    bs_d = pl.BlockSpec((B, D), lambda i: (i, 0))
    bs_c = pl.BlockSpec((B, 1), lambda i: (i, 0))
    bs_w = pl.BlockSpec((D, D), lambda i: (0, 0))
    return pl.pallas_call(
        fbody, grid=grid,
        in_specs=[bs_h, bs_h, bs_c, bs_h, bs_h, bs_c, bs_w, bs_w],
        out_specs=[bs_d, bs_h, bs_h, bs_d, bs_d],
        out_shape=[jax.ShapeDtypeStruct((NI, D), F32),
                   jax.ShapeDtypeStruct((NI, H), F32),
                   jax.ShapeDtypeStruct((NI, H), F32),
                   jax.ShapeDtypeStruct((NI, D), F32),
                   jax.ShapeDtypeStruct((NI, D), F32)],
    )(kg_lo, kg_hi, kg_cnt, in_lo, in_hi, in_cnt, w1, w2)


def _att_div_call(kg_lo, kg_hi, kg_cnt):
    NA = NE - NI
    B = 400
    grid = (NA // B,)

    def abody(kl, kh, kcn, out):
        kg = jnp.concatenate([kl[...], kh[...]], axis=1)
        out[...] = kg / jnp.maximum(kcn[...], 1.0)

    return pl.pallas_call(
        abody, grid=grid,
        in_specs=[pl.BlockSpec((B, H), lambda i: (i, 0)),
                  pl.BlockSpec((B, H), lambda i: (i, 0)),
                  pl.BlockSpec((B, 1), lambda i: (i, 0))],
        out_specs=pl.BlockSpec((B, D), lambda i: (i, 0)),
        out_shape=jax.ShapeDtypeStruct((NA, D), F32),
    )(kg_lo, kg_hi, kg_cnt)


def kernel(entity_emb, user_emb, edge_index, edge_type, interact_mat, weight,
           W1, W2):
    head = edge_index[0]
    tail = edge_index[1]
    row = interact_mat[0]
    col = interact_mat[1]
    ent_lo = entity_emb[:, :H]
    ent_hi = entity_emb[:, H:]
    usr_lo = user_emb[:, :H]
    usr_hi = user_emb[:, H:]
    w_lo = jnp.tile(weight[:, :H], (NTILE, 1))
    w_hi = jnp.tile(weight[:, H:], (NTILE, 1))

    npad = E_PAD - EDG
    zpad = jnp.zeros((npad,), I32)
    trash_ent = (jnp.arange(npad, dtype=I32) % (RP_ENT - NE)) + NE
    trash_itm = (jnp.arange(npad, dtype=I32) % (RP_ITM - NI)) + NI

    head_p = _pad_idx(head, trash_ent)
    tail_p = _pad_idx(tail, zpad)
    type_p = _pad_idx(edge_type, zpad)
    rowg_p = _pad_idx(row, zpad)
    row_p = _pad_idx(row, trash_ent)
    colg_p = _pad_idx(col, zpad)
    col_p = _pad_idx(col, trash_itm)

    ones128 = jnp.ones((SUB,), F32)
    z_ent_rows = jnp.zeros((Q_ENT, H), F32)
    z_ent_cnt = jnp.zeros((RP_ENT,), F32)
    z_itm_rows = jnp.zeros((Q_ITM, H), F32)
    z_itm_cnt = jnp.zeros((RP_ITM,), F32)

    kg_k = _make_sc_kernel(RP_ENT, Q_ENT, 128, 16, True, True)
    kg_lo, kg_hi, kg_cnt = kg_k(ent_lo, ent_hi, tail_p, head_p, type_p,
                                w_lo, w_hi, z_ent_rows, z_ent_cnt, ones128)

    int_k = _make_sc_kernel(RP_ITM, Q_ITM, 512, 10, False, True)
    int_lo, int_hi, int_cnt = int_k(usr_lo, usr_hi, rowg_p, col_p,
                                    z_itm_rows, z_itm_cnt, ones128)

    fus, fus_lo, fus_hi, kg_cat, int_cat = _fusion_call(
        kg_lo[:NI], kg_hi[:NI], kg_cnt[:NI, None],
        int_lo[:NI], int_hi[:NI], int_cnt[:NI, None], W1, W2)

    usr_k = _make_sc_kernel(RP_ENT, Q_ENT, 256, 10, False, False)
    ua_lo, ua_hi = usr_k(fus_lo, fus_hi, colg_p, row_p, z_ent_rows)

    att = _att_div_call(kg_lo[NI:NE], kg_hi[NI:NE], kg_cnt[NI:NE, None])
    final_entity = jnp.concatenate([fus, att], axis=0)
    user_agg = jnp.concatenate([ua_lo[:NU], ua_hi[:NU]], axis=1)
    return final_entity, user_agg, kg_cat, int_cat


# speedup vs baseline: 6.7616x; 1.0181x over previous
import jax
import jax.numpy as jnp
from jax import lax
from jax.experimental import pallas as pl
from jax.experimental.pallas import tpu as pltpu
from jax.experimental.pallas import tpu_sc as plsc

F32 = jnp.float32
I32 = jnp.int32

NE = 50000
NU = 50000
NI = 30000
EDG = 800000
D = 64
H = 32
NREL = 32

NTILE = 16
SUB = 128
E_PAD = 819200
EPT = E_PAD // NTILE
IDXW = 128
IDXROWS = E_PAD // IDXW

RP_ENT = 50176
Q_ENT = 3136
RP_ITM = 30208
Q_ITM = 1888


def _pad_idx(a, pad_vals):
    return jnp.concatenate([a.astype(I32), pad_vals]).reshape(IDXROWS, IDXW)


def _make_sc_kernel(r_pad, q, kch, grp, use_w, mean):
    mesh = plsc.VectorSubcoreMesh(core_axis_name="c", subcore_axis_name="s")
    out_type = [jax.ShapeDtypeStruct((r_pad, H), F32),
                jax.ShapeDtypeStruct((r_pad, H), F32)]
    if mean:
        out_type.append(jax.ShapeDtypeStruct((r_pad,), F32))
    nsub = kch // SUB
    nchunk = EPT // kch
    gn = grp * nsub
    assert grp % 2 == 0 and nchunk % grp == 0
    ngroups = nchunk // grp
    scratch = [
        pltpu.VMEM((gn, IDXW), I32),
        pltpu.VMEM((gn, IDXW), I32),
        pltpu.VMEM((kch, H), F32),
        pltpu.VMEM((kch, H), F32),
        pltpu.VMEM_SHARED((r_pad, H), F32),
        pltpu.SemaphoreType.DMA,
        pltpu.SemaphoreType.DMA,
        pltpu.SemaphoreType.DMA,
    ]
    if use_w:
        scratch += [pltpu.VMEM((gn, IDXW), I32),
                    pltpu.VMEM((kch, H), F32),
                    pltpu.VMEM((kch, H), F32)]
    if mean:
        scratch += [pltpu.VMEM_SHARED((r_pad,), F32),
                    pltpu.VMEM((SUB,), F32)]

    def body(*refs):
        src_lo, src_hi, gidx, sidx = refs[0], refs[1], refs[2], refs[3]
        i = 4
        if use_w:
            tyidx, w_lo, w_hi = refs[i], refs[i + 1], refs[i + 2]
            i += 3
        zrows = refs[i]
        i += 1
        if mean:
            zcnt, ones_h = refs[i], refs[i + 1]
            i += 2
        out_lo, out_hi = refs[i], refs[i + 1]
        i += 2
        if mean:
            out_cnt = refs[i]
            i += 1
        gi_v, si_v, rows0, rows1, acc, sem0, sem1, ssem = refs[i:i + 8]
        i += 8
        if use_w:
            ti_v, wrows0, wrows1 = refs[i], refs[i + 1], refs[i + 2]
            i += 3
        if mean:
            cnt, ones_v = refs[i], refs[i + 1]

        c = lax.axis_index("c")
        rows_b = (rows0, rows1)
        wrows_b = (wrows0, wrows1) if use_w else None
        sems = (sem0, sem1)

        def run(src, wtab, out, write_cnt):
            t = lax.axis_index("s")
            pltpu.sync_copy(zrows, acc.at[pl.ds(t * q, q)])
            if mean:
                @pl.when(t == 0)
                def _():
                    pltpu.sync_copy(zcnt, cnt)
                pltpu.sync_copy(ones_h, ones_v)
            plsc.subcore_barrier()

            def issue_g(ck, pb):
                for j in range(nsub):
                    r = ck * nsub + j
                    pltpu.async_copy(src.at[gi_v.at[r]],
                                     rows_b[pb].at[pl.ds(j * SUB, SUB)],
                                     sems[pb])
                    if use_w:
                        pltpu.async_copy(wtab.at[ti_v.at[r]],
                                         wrows_b[pb].at[pl.ds(j * SUB, SUB)],
                                         sems[pb])

            def wait_g(ck, pb):
                for j in range(nsub):
                    r = ck * nsub + j
                    pltpu.make_async_copy(
                        src.at[gi_v.at[r]],
                        rows_b[pb].at[pl.ds(j * SUB, SUB)], sems[pb]).wait()
                    if use_w:
                        pltpu.make_async_copy(
                            wtab.at[ti_v.at[r]],
                            wrows_b[pb].at[pl.ds(j * SUB, SUB)],
                            sems[pb]).wait()

            def do_chunk(ck, pb):
                wait_g(ck, pb)
                if use_w:
                    def mul16(m, cr):
                        for r8 in range(16):
                            rw = m * 16 + r8
                            for hh in range(2):
                                sl = (rw, pl.ds(hh * 16, 16))
                                rows_b[pb][sl] = rows_b[pb][sl] * wrows_b[pb][sl]
                        return cr
                    lax.fori_loop(0, kch // 16, mul16, 0)
                sds = []
                for j in range(nsub):
                    r = ck * nsub + j
                    sds.append(pltpu.async_copy(
                        rows_b[pb].at[pl.ds(j * SUB, SUB)],
                        acc.at[si_v.at[r]], ssem, add=True))
                    if mean:
                        sds.append(pltpu.async_copy(ones_v, cnt.at[si_v.at[r]],
                                                    ssem, add=True))
                for dd in sds:
                    dd.wait()

            def group(g, carry):
                base = t * (nchunk * nsub) + g * gn
                pltpu.sync_copy(gidx.at[pl.ds(base, gn)], gi_v)
                pltpu.sync_copy(sidx.at[pl.ds(base, gn)], si_v)
                if use_w:
                    pltpu.sync_copy(tyidx.at[pl.ds(base, gn)], ti_v)
                    toff = t * NREL
                    for r in range(gn):
                        for g8 in range(IDXW // 16):
                            sl = (r, pl.ds(g8 * 16, 16))
                            ti_v[sl] = ti_v[sl] + toff
                issue_g(0, 0)

                def pair(p, cr):
                    a = 2 * p
                    issue_g(a + 1, 1)
                    do_chunk(a, 0)

                    @pl.when(p < grp // 2 - 1)
                    def _():
                        issue_g(a + 2, 0)
                    do_chunk(a + 1, 1)
                    return cr

                lax.fori_loop(0, grp // 2, pair, 0)
                return carry

            lax.fori_loop(0, ngroups, group, 0)
            plsc.subcore_barrier()
            pltpu.sync_copy(acc.at[pl.ds(t * q, q)], out.at[pl.ds(t * q, q)])
            if mean and write_cnt:
                @pl.when(t == 0)
                def _():
                    pltpu.sync_copy(cnt, out_cnt)

        @pl.when(c == 0)
        def _():
            run(src_lo, w_lo if use_w else None, out_lo, True)

        @pl.when(c == 1)
        def _():
            run(src_hi, w_hi if use_w else None, out_hi, False)

    return pl.kernel(body, out_type=tuple(out_type), mesh=mesh,
                     scratch_types=scratch,
                     compiler_params=pltpu.CompilerParams(
                         use_tc_tiling_on_sc=False))


def _fusion_call(kg_lo, kg_hi, kg_cnt, in_lo, in_hi, in_cnt, w1, w2):
    B = 600
    grid = (NI // B,)

    def fbody(kl, kh, kcn, il, ih, icn, w1r, w2r, fus, fl, fh, kc, ic):
        kg = jnp.concatenate([kl[...], kh[...]], axis=1)
        kg = kg / jnp.maximum(kcn[...], 1.0)
        it = jnp.concatenate([il[...], ih[...]], axis=1)
        it = it / jnp.maximum(icn[...], 1.0)
        z = lax.dot_general(kg, w1r[...], (((1,), (1,)), ((), ())),
                            preferred_element_type=F32)
        z = z + lax.dot_general(it, w2r[...], (((1,), (1,)), ((), ())),
                                preferred_element_type=F32)
        g = jax.nn.sigmoid(z)
        f = g * kg + (1.0 - g) * it
        fus[...] = f
        fl[...] = f[:, :H]
        fh[...] = f[:, H:]
        kc[...] = kg
        ic[...] = it

    bs_h = pl.BlockSpec((B, H), lambda i: (i, 0))
    bs_d = pl.BlockSpec((B, D), lambda i: (i, 0))
    bs_c = pl.BlockSpec((B, 1), lambda i: (i, 0))
    bs_w = pl.BlockSpec((D, D), lambda i: (0, 0))
    return pl.pallas_call(
        fbody, grid=grid,
        in_specs=[bs_h, bs_h, bs_c, bs_h, bs_h, bs_c, bs_w, bs_w],
        out_specs=[bs_d, bs_h, bs_h, bs_d, bs_d],
        out_shape=[jax.ShapeDtypeStruct((NI, D), F32),
                   jax.ShapeDtypeStruct((NI, H), F32),
                   jax.ShapeDtypeStruct((NI, H), F32),
                   jax.ShapeDtypeStruct((NI, D), F32),
                   jax.ShapeDtypeStruct((NI, D), F32)],
    )(kg_lo, kg_hi, kg_cnt, in_lo, in_hi, in_cnt, w1, w2)


def _att_div_call(kg_lo, kg_hi, kg_cnt):
    NA = NE - NI
    B = 400
    grid = (NA // B,)

    def abody(kl, kh, kcn, out):
        kg = jnp.concatenate([kl[...], kh[...]], axis=1)
        out[...] = kg / jnp.maximum(kcn[...], 1.0)

    return pl.pallas_call(
        abody, grid=grid,
        in_specs=[pl.BlockSpec((B, H), lambda i: (i, 0)),
                  pl.BlockSpec((B, H), lambda i: (i, 0)),
                  pl.BlockSpec((B, 1), lambda i: (i, 0))],
        out_specs=pl.BlockSpec((B, D), lambda i: (i, 0)),
        out_shape=jax.ShapeDtypeStruct((NA, D), F32),
    )(kg_lo, kg_hi, kg_cnt)


def kernel(entity_emb, user_emb, edge_index, edge_type, interact_mat, weight,
           W1, W2):
    head = edge_index[0]
    tail = edge_index[1]
    row = interact_mat[0]
    col = interact_mat[1]
    ent_lo = entity_emb[:, :H]
    ent_hi = entity_emb[:, H:]
    usr_lo = user_emb[:, :H]
    usr_hi = user_emb[:, H:]
    w_lo = jnp.tile(weight[:, :H], (NTILE, 1))
    w_hi = jnp.tile(weight[:, H:], (NTILE, 1))

    npad = E_PAD - EDG
    zpad = jnp.zeros((npad,), I32)
    trash_ent = (jnp.arange(npad, dtype=I32) % (RP_ENT - NE)) + NE
    trash_itm = (jnp.arange(npad, dtype=I32) % (RP_ITM - NI)) + NI

    head_p = _pad_idx(head, trash_ent)
    tail_p = _pad_idx(tail, zpad)
    type_p = _pad_idx(edge_type, zpad)
    rowg_p = _pad_idx(row, zpad)
    row_p = _pad_idx(row, trash_ent)
    colg_p = _pad_idx(col, zpad)
    col_p = _pad_idx(col, trash_itm)

    ones128 = jnp.ones((SUB,), F32)
    z_ent_rows = jnp.zeros((Q_ENT, H), F32)
    z_ent_cnt = jnp.zeros((RP_ENT,), F32)
    z_itm_rows = jnp.zeros((Q_ITM, H), F32)
    z_itm_cnt = jnp.zeros((RP_ITM,), F32)

    kg_k = _make_sc_kernel(RP_ENT, Q_ENT, 128, 16, True, True)
    kg_lo, kg_hi, kg_cnt = kg_k(ent_lo, ent_hi, tail_p, head_p, type_p,
                                w_lo, w_hi, z_ent_rows, z_ent_cnt, ones128)

    int_k = _make_sc_kernel(RP_ITM, Q_ITM, 512, 10, False, True)
    int_lo, int_hi, int_cnt = int_k(usr_lo, usr_hi, rowg_p, col_p,
                                    z_itm_rows, z_itm_cnt, ones128)

    fus, fus_lo, fus_hi, kg_cat, int_cat = _fusion_call(
        kg_lo[:NI], kg_hi[:NI], kg_cnt[:NI, None],
        int_lo[:NI], int_hi[:NI], int_cnt[:NI, None], W1, W2)

    usr_k = _make_sc_kernel(RP_ENT, Q_ENT, 256, 10, False, False)
    ua_lo, ua_hi = usr_k(fus_lo, fus_hi, colg_p, row_p, z_ent_rows)

    att = _att_div_call(kg_lo[NI:NE], kg_hi[NI:NE], kg_cnt[NI:NE, None])
    final_entity = jnp.concatenate([fus, att], axis=0)
    user_agg = jnp.concatenate([ua_lo[:NU], ua_hi[:NU]], axis=1)
    return final_entity, user_agg, kg_cat, int_cat
